# baseline (speedup 1.0000x reference)
import functools
import math

import jax
import jax.numpy as jnp
from jax import lax
from jax.experimental import pallas as pl
from jax.experimental.pallas import tpu as pltpu
from jax.experimental.pallas import tpu_sc as plsc

D_MODEL = 64
CHUNK = 128
NBUF = 2
SCALE = math.sqrt(D_MODEL)


def _make_sc_kernel(n_flat, num_cores, num_subcores):
    n_workers = num_cores * num_subcores
    per_worker = n_flat // n_workers
    n_chunks = per_worker // CHUNK
    n_blocks = n_chunks // NBUF

    mesh = plsc.VectorSubcoreMesh(core_axis_name="c", subcore_axis_name="s")

    @functools.partial(
        pl.kernel,
        mesh=mesh,
        out_type=jax.ShapeDtypeStruct((n_flat, D_MODEL), jnp.float32),
        compiler_params=pltpu.CompilerParams(use_tc_tiling_on_sc=False),
        scratch_types=(
            [pltpu.VMEM((n_chunks, CHUNK), jnp.int32)]
            + [pltpu.VMEM((CHUNK, D_MODEL), jnp.float32)] * (2 * NBUF)
            + [pltpu.SemaphoreType.DMA] * (2 * NBUF)
        ),
    )
    def k(x_hbm, lut_hbm, out_hbm, idx_v, *bufs_and_sems):
        gbuf = bufs_and_sems[0:NBUF]
        obuf = bufs_and_sems[NBUF:2 * NBUF]
        gsem = bufs_and_sems[2 * NBUF:3 * NBUF]
        osem = bufs_and_sems[3 * NBUF:4 * NBUF]

        wid = lax.axis_index("s") * num_cores + lax.axis_index("c")
        pltpu.sync_copy(x_hbm.at[pl.ds(wid * n_chunks, n_chunks)], idx_v)
        out_base = wid * per_worker

        def start_gather(c, b):
            pltpu.async_copy(lut_hbm.at[idx_v.at[c]], gbuf[b], gsem[b])

        def wait_gather(c, b):
            pltpu.make_async_copy(lut_hbm.at[idx_v.at[c]], gbuf[b],
                                  gsem[b]).wait()

        def start_out(c, b):
            dst = out_hbm.at[pl.ds(out_base + c * CHUNK, CHUNK)]
            pltpu.async_copy(obuf[b], dst, osem[b])

        def wait_out(c, b):
            dst = out_hbm.at[pl.ds(out_base + c * CHUNK, CHUNK)]
            pltpu.make_async_copy(obuf[b], dst, osem[b]).wait()

        def scale(b):
            gb, ob = gbuf[b], obuf[b]

            def row_body(r, rc):
                for d in range(D_MODEL // 16):
                    sl = pl.ds(d * 16, 16)
                    ob[r, sl] = gb[r, sl] * SCALE
                return rc

            lax.fori_loop(0, CHUNK, row_body, 0, unroll=4)

        for b in range(NBUF):
            start_gather(b, b)
        for b in range(NBUF):
            wait_gather(b, b)
            scale(b)
            start_out(b, b)
            start_gather(NBUF + b, b)

        def block_body(kk, carry):
            for b in range(NBUF):
                c = kk * NBUF + b
                wait_gather(c, b)
                wait_out(c - NBUF, b)
                scale(b)
                start_out(c, b)
                start_gather(c + NBUF, b)
            return carry

        lax.fori_loop(1, n_blocks - 1, block_body, 0)

        for b in range(NBUF):
            c = (n_blocks - 1) * NBUF + b
            wait_gather(c, b)
            wait_out(c - NBUF, b)
            scale(b)
            start_out(c, b)
        for b in range(NBUF):
            wait_out((n_blocks - 1) * NBUF + b, b)

    return k


def kernel(x, lut):
    xf = x.reshape(-1).astype(jnp.int32)
    n_flat = xf.shape[0]
    info = plsc.get_sparse_core_info()
    x2d = xf.reshape(n_flat // CHUNK, CHUNK)
    out = _make_sc_kernel(n_flat, info.num_cores, info.num_subcores)(x2d, lut)
    return out.reshape(*x.shape, D_MODEL)


# speedup vs baseline: 1.2604x; 1.2604x over previous
import functools
import math

import jax
import jax.numpy as jnp
from jax import lax
from jax.experimental import pallas as pl
from jax.experimental.pallas import tpu as pltpu
from jax.experimental.pallas import tpu_sc as plsc

D_MODEL = 64
ROW = 200
SPLIT = 128
NBUF = 2
SCALE = math.sqrt(D_MODEL)


def _make_sc_kernel(n_rows, num_cores, num_subcores):
    n_workers = num_cores * num_subcores
    rows_per_worker = n_rows // n_workers
    n_blocks = rows_per_worker // NBUF

    mesh = plsc.VectorSubcoreMesh(core_axis_name="c", subcore_axis_name="s")

    @functools.partial(
        pl.kernel,
        mesh=mesh,
        out_type=jax.ShapeDtypeStruct((n_rows, ROW, D_MODEL), jnp.float32),
        compiler_params=pltpu.CompilerParams(use_tc_tiling_on_sc=False),
        scratch_types=(
            [pltpu.VMEM((rows_per_worker, ROW), jnp.int32)]
            + [pltpu.VMEM((ROW, D_MODEL), jnp.float32)] * (2 * NBUF)
            + [pltpu.SemaphoreType.DMA] * (2 * NBUF)
        ),
    )
    def k(x_hbm, lut_hbm, out_hbm, idx_v, *bufs_and_sems):
        gbuf = bufs_and_sems[0:NBUF]
        obuf = bufs_and_sems[NBUF:2 * NBUF]
        gsem = bufs_and_sems[2 * NBUF:3 * NBUF]
        osem = bufs_and_sems[3 * NBUF:4 * NBUF]

        wid = lax.axis_index("s") * num_cores + lax.axis_index("c")
        row0 = wid * rows_per_worker
        pltpu.sync_copy(x_hbm.at[pl.ds(row0, rows_per_worker)], idx_v)

        def start_gather(r, b):
            pltpu.async_copy(lut_hbm.at[idx_v.at[r, pl.ds(0, SPLIT)]],
                             gbuf[b].at[pl.ds(0, SPLIT)], gsem[b])
            pltpu.async_copy(lut_hbm.at[idx_v.at[r, pl.ds(SPLIT, ROW - SPLIT)]],
                             gbuf[b].at[pl.ds(SPLIT, ROW - SPLIT)], gsem[b])

        def wait_gather(b):
            pltpu.make_async_copy(lut_hbm.at[pl.ds(0, ROW)], gbuf[b],
                                  gsem[b]).wait()

        def start_out(r, b):
            pltpu.async_copy(obuf[b], out_hbm.at[row0 + r], osem[b])

        def wait_out(b):
            pltpu.make_async_copy(obuf[b], out_hbm.at[row0], osem[b]).wait()

        def scale(b):
            gb, ob = gbuf[b], obuf[b]

            def body(j, carry):
                base = j * 8
                vals = []
                for kk in range(8):
                    for d in range(D_MODEL // 16):
                        vals.append(gb[base + kk, pl.ds(d * 16, 16)])
                i = 0
                for kk in range(8):
                    for d in range(D_MODEL // 16):
                        ob[base + kk, pl.ds(d * 16, 16)] = vals[i] * SCALE
                        i += 1
                return carry

            lax.fori_loop(0, ROW // 8, body, 0)

        for b in range(NBUF):
            start_gather(b, b)
        for b in range(NBUF):
            wait_gather(b)
            scale(b)
            start_out(b, b)
            start_gather(NBUF + b, b)

        def block_body(kk, carry):
            for b in range(NBUF):
                r = kk * NBUF + b
                wait_gather(b)
                wait_out(b)
                scale(b)
                start_out(r, b)
                start_gather(r + NBUF, b)
            return carry

        lax.fori_loop(1, n_blocks - 1, block_body, 0)

        for b in range(NBUF):
            r = (n_blocks - 1) * NBUF + b
            wait_gather(b)
            wait_out(b)
            scale(b)
            start_out(r, b)
        for b in range(NBUF):
            wait_out(b)

    return k


def kernel(x, lut):
    xi = x.astype(jnp.int32)
    info = plsc.get_sparse_core_info()
    return _make_sc_kernel(x.shape[0], info.num_cores, info.num_subcores)(
        xi, lut)


# speedup vs baseline: 1.7996x; 1.4278x over previous
import functools
import math

import jax
import jax.numpy as jnp
from jax import lax
from jax.experimental import pallas as pl
from jax.experimental.pallas import tpu as pltpu
from jax.experimental.pallas import tpu_sc as plsc

D_MODEL = 64
ROW = 200
SPLIT = 128
NBUF = 2
SCALE = math.sqrt(D_MODEL)


def _make_sc_kernel(n_rows, num_cores, num_subcores):
    n_workers = num_cores * num_subcores
    rows_per_worker = n_rows // n_workers
    n_blocks = rows_per_worker // NBUF

    mesh = plsc.VectorSubcoreMesh(core_axis_name="c", subcore_axis_name="s")

    @functools.partial(
        pl.kernel,
        mesh=mesh,
        out_type=jax.ShapeDtypeStruct((n_rows * ROW, 128), jnp.float32),
        compiler_params=pltpu.CompilerParams(use_tc_tiling_on_sc=False),
        scratch_types=(
            [pltpu.VMEM((rows_per_worker, ROW), jnp.int32)]
            + [pltpu.VMEM((ROW, D_MODEL), jnp.float32)] * (2 * NBUF)
            + [pltpu.SemaphoreType.DMA] * (2 * NBUF)
        ),
    )
    def k(x_hbm, lut_hbm, out_hbm, idx_v, *bufs_and_sems):
        gbuf = bufs_and_sems[0:NBUF]
        obuf = bufs_and_sems[NBUF:2 * NBUF]
        gsem = bufs_and_sems[2 * NBUF:3 * NBUF]
        osem = bufs_and_sems[3 * NBUF:4 * NBUF]

        wid = lax.axis_index("s") * num_cores + lax.axis_index("c")
        row0 = wid * rows_per_worker
        pltpu.sync_copy(x_hbm.at[pl.ds(row0, rows_per_worker)], idx_v)

        def start_gather(r, b):
            pltpu.async_copy(lut_hbm.at[idx_v.at[r, pl.ds(0, SPLIT)]],
                             gbuf[b].at[pl.ds(0, SPLIT)], gsem[b])
            pltpu.async_copy(lut_hbm.at[idx_v.at[r, pl.ds(SPLIT, ROW - SPLIT)]],
                             gbuf[b].at[pl.ds(SPLIT, ROW - SPLIT)], gsem[b])

        def wait_gather(b):
            pltpu.make_async_copy(lut_hbm.at[pl.ds(0, ROW)], gbuf[b],
                                  gsem[b]).wait()

        def start_out(r, b):
            dst = out_hbm.at[pl.ds((row0 + r) * ROW, ROW), pl.ds(0, D_MODEL)]
            pltpu.async_copy(obuf[b], dst, osem[b])

        def wait_out(b):
            dst = out_hbm.at[pl.ds(0, ROW), pl.ds(0, D_MODEL)]
            pltpu.make_async_copy(obuf[b], dst, osem[b]).wait()

        def scale(b):
            gb, ob = gbuf[b], obuf[b]

            def body(j, carry):
                base = j * 8
                vals = []
                for kk in range(8):
                    for d in range(D_MODEL // 16):
                        vals.append(gb[base + kk, pl.ds(d * 16, 16)])
                i = 0
                for kk in range(8):
                    for d in range(D_MODEL // 16):
                        ob[base + kk, pl.ds(d * 16, 16)] = vals[i] * SCALE
                        i += 1
                return carry

            lax.fori_loop(0, ROW // 8, body, 0)

        for b in range(NBUF):
            start_gather(b, b)
        for b in range(NBUF):
            wait_gather(b)
            scale(b)
            start_out(b, b)
            start_gather(NBUF + b, b)

        def block_body(kk, carry):
            for b in range(NBUF):
                r = kk * NBUF + b
                wait_gather(b)
                wait_out(b)
                scale(b)
                start_out(r, b)
                start_gather(r + NBUF, b)
            return carry

        lax.fori_loop(1, n_blocks - 1, block_body, 0)

        for b in range(NBUF):
            r = (n_blocks - 1) * NBUF + b
            wait_gather(b)
            wait_out(b)
            scale(b)
            start_out(r, b)
        for b in range(NBUF):
            wait_out(b)

    return k


def kernel(x, lut):
    xi = x.astype(jnp.int32) * 2
    lut_pad = jnp.pad(lut, ((0, 0), (0, 128 - D_MODEL)))
    lut2d = lut_pad.reshape(2 * lut.shape[0], D_MODEL)
    info = plsc.get_sparse_core_info()
    out128 = _make_sc_kernel(x.shape[0], info.num_cores, info.num_subcores)(
        xi, lut2d)
    return out128[:, :D_MODEL].reshape(x.shape[0], x.shape[1], D_MODEL)


# speedup vs baseline: 1.8009x; 1.0007x over previous
import functools
import math

import jax
import jax.numpy as jnp
from jax import lax
from jax.experimental import pallas as pl
from jax.experimental.pallas import tpu as pltpu
from jax.experimental.pallas import tpu_sc as plsc

D_MODEL = 64
ROW = 200
SPLIT = 128
NBUF = 2
SCALE = math.sqrt(D_MODEL)


def _make_sc_kernel(n_rows, num_cores, num_subcores):
    n_workers = num_cores * num_subcores
    rows_per_worker = n_rows // n_workers
    n_blocks = rows_per_worker // NBUF

    mesh = plsc.VectorSubcoreMesh(core_axis_name="c", subcore_axis_name="s")

    @functools.partial(
        pl.kernel,
        mesh=mesh,
        out_type=jax.ShapeDtypeStruct((n_rows * ROW, 128), jnp.float32),
        compiler_params=pltpu.CompilerParams(use_tc_tiling_on_sc=False),
        scratch_types=(
            [pltpu.VMEM((rows_per_worker, ROW), jnp.int32)]
            + [pltpu.VMEM((ROW, D_MODEL), jnp.float32)] * (2 * NBUF)
            + [pltpu.SemaphoreType.DMA] * (2 * NBUF)
        ),
    )
    def k(x_hbm, lut_hbm, out_hbm, idx_v, *bufs_and_sems):
        gbuf = bufs_and_sems[0:NBUF]
        obuf = bufs_and_sems[NBUF:2 * NBUF]
        gsem = bufs_and_sems[2 * NBUF:3 * NBUF]
        osem = bufs_and_sems[3 * NBUF:4 * NBUF]

        wid = lax.axis_index("s") * num_cores + lax.axis_index("c")
        row0 = wid * rows_per_worker
        pltpu.sync_copy(x_hbm.at[pl.ds(row0, rows_per_worker)], idx_v)

        def start_gather(r, b):
            pltpu.async_copy(lut_hbm.at[idx_v.at[r, pl.ds(0, SPLIT)]],
                             gbuf[b].at[pl.ds(0, SPLIT)], gsem[b])
            pltpu.async_copy(lut_hbm.at[idx_v.at[r, pl.ds(SPLIT, ROW - SPLIT)]],
                             gbuf[b].at[pl.ds(SPLIT, ROW - SPLIT)], gsem[b])

        def wait_gather(b):
            pltpu.make_async_copy(lut_hbm.at[pl.ds(0, ROW)], gbuf[b],
                                  gsem[b]).wait()

        def start_out(r, b):
            dst = out_hbm.at[pl.ds((row0 + r) * ROW, ROW), pl.ds(0, D_MODEL)]
            pltpu.async_copy(obuf[b], dst, osem[b])

        def wait_out(b):
            dst = out_hbm.at[pl.ds(0, ROW), pl.ds(0, D_MODEL)]
            pltpu.make_async_copy(obuf[b], dst, osem[b]).wait()

        def scale(b):
            gb, ob = gbuf[b], obuf[b]

            def body(j, carry):
                base = j * 8
                vals = []
                for kk in range(8):
                    for d in range(D_MODEL // 16):
                        vals.append(gb[base + kk, pl.ds(d * 16, 16)])
                i = 0
                for kk in range(8):
                    for d in range(D_MODEL // 16):
                        ob[base + kk, pl.ds(d * 16, 16)] = vals[i] * SCALE
                        i += 1
                return carry

            lax.fori_loop(0, ROW // 8, body, 0)

        for b in range(NBUF):
            start_gather(b, b)
        for b in range(NBUF):
            wait_gather(b)
            scale(b)
            start_out(b, b)
            start_gather(NBUF + b, b)

        def block_body(kk, carry):
            for b in range(NBUF):
                r = kk * NBUF + b
                wait_gather(b)
                wait_out(b)
                scale(b)
                start_out(r, b)
                start_gather(r + NBUF, b)
            return carry

        lax.fori_loop(1, n_blocks - 1, block_body, 0)

        for b in range(NBUF):
            r = (n_blocks - 1) * NBUF + b
            wait_gather(b)
            wait_out(b)
            scale(b)
            start_out(r, b)
        for b in range(NBUF):
            wait_out(b)

    return k


def kernel(x, lut):
    xi = x.astype(jnp.int32) * 2
    lut_pad = jnp.pad(jax.lax.optimization_barrier(lut),
                      ((0, 0), (0, 128 - D_MODEL)))
    lut2d = lut_pad.reshape(2 * lut.shape[0], D_MODEL)
    info = plsc.get_sparse_core_info()
    out128 = _make_sc_kernel(x.shape[0], info.num_cores, info.num_subcores)(
        xi, lut2d)
    return out128[:, :D_MODEL].reshape(x.shape[0], x.shape[1], D_MODEL)


# speedup vs baseline: 2.2730x; 1.2621x over previous
import functools
import math

import jax
import jax.numpy as jnp
from jax import lax
from jax.experimental import pallas as pl
from jax.experimental.pallas import tpu as pltpu
from jax.experimental.pallas import tpu_sc as plsc

D_MODEL = 64
ROW = 200
SPLIT = 128
NBUF = 2
SCALE = math.sqrt(D_MODEL)


def _make_sc_kernel(n_rows, num_cores, num_subcores):
    n_workers = num_cores * num_subcores
    rows_per_worker = n_rows // n_workers
    n_blocks = rows_per_worker // NBUF

    mesh = plsc.VectorSubcoreMesh(core_axis_name="c", subcore_axis_name="s")

    @functools.partial(
        pl.kernel,
        mesh=mesh,
        out_type=jax.ShapeDtypeStruct((n_rows * ROW, 128), jnp.float32),
        compiler_params=pltpu.CompilerParams(use_tc_tiling_on_sc=False),
        scratch_types=(
            [pltpu.VMEM((rows_per_worker, ROW), jnp.int32)]
            + [pltpu.VMEM((ROW, D_MODEL), jnp.float32)] * (2 * NBUF)
            + [pltpu.SemaphoreType.DMA] * (2 * NBUF)
        ),
    )
    def k(x_hbm, lut_hbm, out_hbm, idx_v, *bufs_and_sems):
        gbuf = bufs_and_sems[0:NBUF]
        obuf = bufs_and_sems[NBUF:2 * NBUF]
        gsem = bufs_and_sems[2 * NBUF:3 * NBUF]
        osem = bufs_and_sems[3 * NBUF:4 * NBUF]

        wid = lax.axis_index("s") * num_cores + lax.axis_index("c")
        row0 = wid * rows_per_worker
        pltpu.sync_copy(x_hbm.at[pl.ds(row0, rows_per_worker)], idx_v)

        def start_gather(r, b):
            pltpu.async_copy(lut_hbm.at[idx_v.at[r, pl.ds(0, SPLIT)]],
                             gbuf[b].at[pl.ds(0, SPLIT)], gsem[b])
            pltpu.async_copy(lut_hbm.at[idx_v.at[r, pl.ds(SPLIT, ROW - SPLIT)]],
                             gbuf[b].at[pl.ds(SPLIT, ROW - SPLIT)], gsem[b])

        def wait_gather(b):
            pltpu.make_async_copy(lut_hbm.at[pl.ds(0, ROW)], gbuf[b],
                                  gsem[b]).wait()

        def start_out(r, b):
            dst = out_hbm.at[pl.ds((row0 + r) * ROW, ROW), pl.ds(0, D_MODEL)]
            pltpu.async_copy(obuf[b], dst, osem[b])

        def wait_out(b):
            dst = out_hbm.at[pl.ds(0, ROW), pl.ds(0, D_MODEL)]
            pltpu.make_async_copy(obuf[b], dst, osem[b]).wait()

        def scale(b):
            gb, ob = gbuf[b], obuf[b]

            def body(j, carry):
                base = j * 8
                vals = []
                for kk in range(8):
                    for d in range(D_MODEL // 16):
                        vals.append(gb[base + kk, pl.ds(d * 16, 16)])
                i = 0
                for kk in range(8):
                    for d in range(D_MODEL // 16):
                        ob[base + kk, pl.ds(d * 16, 16)] = vals[i] * SCALE
                        i += 1
                return carry

            lax.fori_loop(0, ROW // 8, body, 0)

        for b in range(NBUF):
            start_gather(b, b)
        for b in range(NBUF):
            wait_gather(b)
            scale(b)
            start_out(b, b)
            start_gather(NBUF + b, b)

        def block_body(kk, carry):
            for b in range(NBUF):
                r = kk * NBUF + b
                wait_gather(b)
                wait_out(b)
                scale(b)
                start_out(r, b)
                start_gather(r + NBUF, b)
            return carry

        lax.fori_loop(1, n_blocks - 1, block_body, 0)

        for b in range(NBUF):
            r = (n_blocks - 1) * NBUF + b
            wait_gather(b)
            wait_out(b)
            scale(b)
            start_out(r, b)
        for b in range(NBUF):
            wait_out(b)

    return k


TBLK = 4096


def _make_tc_transpose(n_vocab):
    grid = (n_vocab + TBLK - 1) // TBLK

    def body(lutt_ref, out_ref):
        out_ref[:, 0:D_MODEL] = lutt_ref[...].T

    return pl.pallas_call(
        body,
        grid=(grid,),
        in_specs=[pl.BlockSpec((D_MODEL, TBLK), lambda i: (0, i))],
        out_specs=pl.BlockSpec((TBLK, 128), lambda i: (i, 0)),
        out_shape=jax.ShapeDtypeStruct((n_vocab, 128), jnp.float32),
    )


def kernel(x, lut):
    xi = x.astype(jnp.int32) * 2
    lut_pad = _make_tc_transpose(lut.shape[0])(jnp.transpose(lut))
    lut2d = lut_pad.reshape(2 * lut.shape[0], D_MODEL)
    info = plsc.get_sparse_core_info()
    out128 = _make_sc_kernel(x.shape[0], info.num_cores, info.num_subcores)(
        xi, lut2d)
    return out128[:, :D_MODEL].reshape(x.shape[0], x.shape[1], D_MODEL)


# speedup vs baseline: 2.6326x; 1.1582x over previous
import functools
import math

import jax
import jax.numpy as jnp
from jax import lax
from jax.experimental import pallas as pl
from jax.experimental.pallas import tpu as pltpu
from jax.experimental.pallas import tpu_sc as plsc

D_MODEL = 64
ROW = 200
SPLIT = 128
NBUF = 2
SCALE = math.sqrt(D_MODEL)


def _make_sc_kernel(n_rows, num_cores, num_subcores):
    n_workers = num_cores * num_subcores
    rows_per_worker = n_rows // n_workers
    n_blocks = rows_per_worker // NBUF

    mesh = plsc.VectorSubcoreMesh(core_axis_name="c", subcore_axis_name="s")

    @functools.partial(
        pl.kernel,
        mesh=mesh,
        out_type=jax.ShapeDtypeStruct((n_rows * ROW, 128), jnp.float32),
        compiler_params=pltpu.CompilerParams(use_tc_tiling_on_sc=False),
        scratch_types=(
            [pltpu.VMEM((rows_per_worker, ROW), jnp.int32)]
            + [pltpu.VMEM((ROW, D_MODEL), jnp.float32)] * (2 * NBUF)
            + [pltpu.SemaphoreType.DMA] * (2 * NBUF)
        ),
    )
    def k(x_hbm, lut_hbm, out_hbm, idx_v, *bufs_and_sems):
        gbuf = bufs_and_sems[0:NBUF]
        obuf = bufs_and_sems[NBUF:2 * NBUF]
        gsem = bufs_and_sems[2 * NBUF:3 * NBUF]
        osem = bufs_and_sems[3 * NBUF:4 * NBUF]

        wid = lax.axis_index("s") * num_cores + lax.axis_index("c")
        row0 = wid * rows_per_worker
        pltpu.sync_copy(x_hbm.at[pl.ds(row0, rows_per_worker)], idx_v)

        def start_gather(r, b):
            pltpu.async_copy(lut_hbm.at[idx_v.at[r, pl.ds(0, SPLIT)]],
                             gbuf[b].at[pl.ds(0, SPLIT)], gsem[b])
            pltpu.async_copy(lut_hbm.at[idx_v.at[r, pl.ds(SPLIT, ROW - SPLIT)]],
                             gbuf[b].at[pl.ds(SPLIT, ROW - SPLIT)], gsem[b])

        def wait_gather(b):
            pltpu.make_async_copy(lut_hbm.at[pl.ds(0, ROW)], gbuf[b],
                                  gsem[b]).wait()

        def start_out(r, b):
            dst = out_hbm.at[pl.ds((row0 + r) * ROW, ROW), pl.ds(0, D_MODEL)]
            pltpu.async_copy(obuf[b], dst, osem[b])

        def wait_out(b):
            dst = out_hbm.at[pl.ds(0, ROW), pl.ds(0, D_MODEL)]
            pltpu.make_async_copy(obuf[b], dst, osem[b]).wait()

        def scale(b):
            gb, ob = gbuf[b], obuf[b]

            def body(j, carry):
                base = j * 8
                vals = []
                for kk in range(8):
                    for d in range(D_MODEL // 16):
                        vals.append(gb[base + kk, pl.ds(d * 16, 16)])
                i = 0
                for kk in range(8):
                    for d in range(D_MODEL // 16):
                        ob[base + kk, pl.ds(d * 16, 16)] = vals[i] * SCALE
                        i += 1
                return carry

            lax.fori_loop(0, ROW // 8, body, 0)

        for b in range(NBUF):
            start_gather(b, b)
        for b in range(NBUF):
            wait_gather(b)
            scale(b)
            start_out(b, b)
            start_gather(NBUF + b, b)

        def block_body(kk, carry):
            for b in range(NBUF):
                r = kk * NBUF + b
                wait_gather(b)
                wait_out(b)
                scale(b)
                start_out(r, b)
                start_gather(r + NBUF, b)
            return carry

        lax.fori_loop(1, n_blocks - 1, block_body, 0)

        for b in range(NBUF):
            r = (n_blocks - 1) * NBUF + b
            wait_gather(b)
            wait_out(b)
            scale(b)
            start_out(r, b)
        for b in range(NBUF):
            wait_out(b)

    return k


TBLK = 16384


def _make_tc_transpose(n_vocab):
    grid = (n_vocab + TBLK - 1) // TBLK

    def body(lutt_ref, out_ref):
        out_ref[:, 0:D_MODEL] = lutt_ref[...].T

    return pl.pallas_call(
        body,
        grid=(grid,),
        in_specs=[pl.BlockSpec((D_MODEL, TBLK), lambda i: (0, i))],
        out_specs=pl.BlockSpec((TBLK, 128), lambda i: (i, 0)),
        out_shape=jax.ShapeDtypeStruct((n_vocab, 128), jnp.float32),
    )


def kernel(x, lut):
    xi = x.astype(jnp.int32) * 2
    lut_pad = _make_tc_transpose(lut.shape[0])(jnp.transpose(lut))
    lut2d = lut_pad.reshape(2 * lut.shape[0], D_MODEL)
    info = plsc.get_sparse_core_info()
    out128 = _make_sc_kernel(x.shape[0], info.num_cores, info.num_subcores)(
        xi, lut2d)
    return out128[:, :D_MODEL].reshape(x.shape[0], x.shape[1], D_MODEL)


# speedup vs baseline: 2.6823x; 1.0188x over previous
import functools
import math

import jax
import jax.numpy as jnp
from jax import lax
from jax.experimental import pallas as pl
from jax.experimental.pallas import tpu as pltpu
from jax.experimental.pallas import tpu_sc as plsc

D_MODEL = 64
ROW = 200
SPLIT = 128
NBUF = 3
SCALE = math.sqrt(D_MODEL)


def _make_sc_kernel(n_rows, num_cores, num_subcores):
    n_workers = num_cores * num_subcores
    rows_per_worker = n_rows // n_workers
    n_blocks = rows_per_worker // NBUF

    mesh = plsc.VectorSubcoreMesh(core_axis_name="c", subcore_axis_name="s")

    @functools.partial(
        pl.kernel,
        mesh=mesh,
        out_type=jax.ShapeDtypeStruct((n_rows * ROW, 128), jnp.float32),
        compiler_params=pltpu.CompilerParams(use_tc_tiling_on_sc=False),
        scratch_types=(
            [pltpu.VMEM((rows_per_worker, ROW), jnp.int32)]
            + [pltpu.VMEM((ROW, D_MODEL), jnp.float32)] * (2 * NBUF)
            + [pltpu.SemaphoreType.DMA] * (2 * NBUF)
        ),
    )
    def k(x_hbm, lut_hbm, out_hbm, idx_v, *bufs_and_sems):
        gbuf = bufs_and_sems[0:NBUF]
        obuf = bufs_and_sems[NBUF:2 * NBUF]
        gsem = bufs_and_sems[2 * NBUF:3 * NBUF]
        osem = bufs_and_sems[3 * NBUF:4 * NBUF]

        wid = lax.axis_index("s") * num_cores + lax.axis_index("c")
        row0 = wid * rows_per_worker
        pltpu.sync_copy(x_hbm.at[pl.ds(row0, rows_per_worker)], idx_v)

        def start_gather(r, b):
            pltpu.async_copy(lut_hbm.at[idx_v.at[r, pl.ds(0, SPLIT)]],
                             gbuf[b].at[pl.ds(0, SPLIT)], gsem[b])
            pltpu.async_copy(lut_hbm.at[idx_v.at[r, pl.ds(SPLIT, ROW - SPLIT)]],
                             gbuf[b].at[pl.ds(SPLIT, ROW - SPLIT)], gsem[b])

        def wait_gather(b):
            pltpu.make_async_copy(lut_hbm.at[pl.ds(0, ROW)], gbuf[b],
                                  gsem[b]).wait()

        def start_out(r, b):
            dst = out_hbm.at[pl.ds((row0 + r) * ROW, ROW), pl.ds(0, D_MODEL)]
            pltpu.async_copy(obuf[b], dst, osem[b])

        def wait_out(b):
            dst = out_hbm.at[pl.ds(0, ROW), pl.ds(0, D_MODEL)]
            pltpu.make_async_copy(obuf[b], dst, osem[b]).wait()

        def scale(b):
            gb, ob = gbuf[b], obuf[b]

            def body(j, carry):
                base = j * 8
                vals = []
                for kk in range(8):
                    for d in range(D_MODEL // 16):
                        vals.append(gb[base + kk, pl.ds(d * 16, 16)])
                i = 0
                for kk in range(8):
                    for d in range(D_MODEL // 16):
                        ob[base + kk, pl.ds(d * 16, 16)] = vals[i] * SCALE
                        i += 1
                return carry

            lax.fori_loop(0, ROW // 8, body, 0)

        for b in range(NBUF):
            start_gather(b, b)
        for b in range(NBUF):
            wait_gather(b)
            scale(b)
            start_out(b, b)
            start_gather(NBUF + b, b)

        def block_body(kk, carry):
            for b in range(NBUF):
                r = kk * NBUF + b
                wait_gather(b)
                wait_out(b)
                scale(b)
                start_out(r, b)
                start_gather(r + NBUF, b)
            return carry

        lax.fori_loop(1, n_blocks - 1, block_body, 0)

        for b in range(NBUF):
            r = (n_blocks - 1) * NBUF + b
            wait_gather(b)
            wait_out(b)
            scale(b)
            start_out(r, b)
        for b in range(NBUF):
            wait_out(b)

    return k


TBLK = 32768


def _make_tc_transpose(n_vocab):
    grid = (n_vocab + TBLK - 1) // TBLK

    def body(lutt_ref, out_ref):
        out_ref[:, 0:D_MODEL] = lutt_ref[...].T

    return pl.pallas_call(
        body,
        grid=(grid,),
        in_specs=[pl.BlockSpec((D_MODEL, TBLK), lambda i: (0, i))],
        out_specs=pl.BlockSpec((TBLK, 128), lambda i: (i, 0)),
        out_shape=jax.ShapeDtypeStruct((n_vocab, 128), jnp.float32),
    )


def kernel(x, lut):
    xi = x.astype(jnp.int32) * 2
    lut_pad = _make_tc_transpose(lut.shape[0])(jnp.transpose(lut))
    lut2d = lut_pad.reshape(2 * lut.shape[0], D_MODEL)
    info = plsc.get_sparse_core_info()
    out128 = _make_sc_kernel(x.shape[0], info.num_cores, info.num_subcores)(
        xi, lut2d)
    return out128[:, :D_MODEL].reshape(x.shape[0], x.shape[1], D_MODEL)
